# baseline (speedup 1.0000x reference)
import functools

import jax
import jax.numpy as jnp
from jax.experimental import pallas as pl
from jax.experimental.pallas import tpu as pltpu
from jax.experimental.pallas import tpu_sc as plsc

D = 1024
E = 8
K = 2
H = 4096
M = 512
NB = 23
NRP = NB * M
HB = 512
NH = H // HB


def _ffn_kernel(be_ref, na_ref, xg_ref, gate_ref, up_ref, down_ref, w_ref,
                o_ref, acc_ref):
    b = pl.program_id(0)
    h = pl.program_id(1)

    @pl.when(b < na_ref[0])
    def _():
        xg = xg_ref[...]
        g = jax.lax.dot_general(
            xg, gate_ref[0].astype(jnp.bfloat16), (((1,), (1,)), ((), ())),
            preferred_element_type=jnp.float32)
        u = jax.lax.dot_general(
            xg, up_ref[0].astype(jnp.bfloat16), (((1,), (1,)), ((), ())),
            preferred_element_type=jnp.float32)
        hact = (g * jax.nn.sigmoid(g) * u).astype(jnp.bfloat16)
        yp = jax.lax.dot_general(
            hact, down_ref[0].astype(jnp.bfloat16), (((1,), (1,)), ((), ())),
            preferred_element_type=jnp.float32)

        @pl.when(h == 0)
        def _():
            acc_ref[...] = yp

        @pl.when(h > 0)
        def _():
            acc_ref[...] += yp

        @pl.when(h == NH - 1)
        def _():
            o_ref[...] = acc_ref[...] * w_ref[...]


def _grouped_ffn(block_expert, nact, xg, gate_W, up_W, down_W, w_col):
    def _row_ix(b, h, be, na):
        return (jnp.where(b < na[0], b, jnp.minimum(na[0], NB - 1)), 0)

    def _gu_ix(b, h, be, na):
        return (be[b], jnp.where(b < na[0], h, 0), 0)

    def _dn_ix(b, h, be, na):
        return (be[b], 0, jnp.where(b < na[0], h, 0))

    grid_spec = pltpu.PrefetchScalarGridSpec(
        num_scalar_prefetch=2,
        grid=(NB, NH),
        in_specs=[
            pl.BlockSpec((M, D), _row_ix),
            pl.BlockSpec((1, HB, D), _gu_ix),
            pl.BlockSpec((1, HB, D), _gu_ix),
            pl.BlockSpec((1, D, HB), _dn_ix),
            pl.BlockSpec((M, 1), _row_ix),
        ],
        out_specs=pl.BlockSpec((M, D), _row_ix),
        scratch_shapes=[pltpu.VMEM((M, D), jnp.float32)],
    )
    return pl.pallas_call(
        _ffn_kernel,
        grid_spec=grid_spec,
        out_shape=jax.ShapeDtypeStruct((NRP, D), jnp.float32),
    )(block_expert, nact, xg, gate_W, up_W, down_W, w_col)


GW = 128
DT = 128
NDT = D // DT


def _sc_combine(yw, r1, r2, n_tok):
    ywf = yw.reshape(NRP * NDT, DT)
    cid = jnp.arange(NDT, dtype=jnp.int32)[:, None]
    r1c = r1[None, :] * NDT + cid
    r2c = r2[None, :] * NDT + cid
    mesh = plsc.VectorSubcoreMesh(core_axis_name="c", subcore_axis_name="s")

    @functools.partial(
        pl.kernel,
        out_type=jax.ShapeDtypeStruct((n_tok, D), jnp.float32),
        mesh=mesh,
        scratch_types=[pltpu.VMEM((GW, DT), jnp.float32),
                       pltpu.VMEM((GW, DT), jnp.float32)])
    def combine_kernel(ywf_hbm, r1_hbm, r2_hbm, o_hbm, g1, g2):
        def body(r1_vmem, r2_vmem, o_vmem):
            pltpu.sync_copy(ywf_hbm.at[r1_vmem.at[0]], g1)
            pltpu.sync_copy(ywf_hbm.at[r2_vmem.at[0]], g2)

            @pl.loop(0, GW)
            def _(i):
                @pl.loop(0, DT, step=16)
                def _(j):
                    slc = (pl.ds(i, 1), pl.ds(j, 16))
                    o_vmem.at[slc][...] = g1.at[slc][...] + g2.at[slc][...]

        pltpu.emit_pipeline(
            body,
            grid=(n_tok // GW, NDT),
            in_specs=[pl.BlockSpec((1, GW), lambda i, c: (c, i)),
                      pl.BlockSpec((1, GW), lambda i, c: (c, i))],
            out_specs=[pl.BlockSpec((GW, DT), lambda i, c: (i, c))],
            core_axis_name=("c", "s"),
            dimension_semantics=(pltpu.PARALLEL, pltpu.PARALLEL),
        )(r1_hbm, r2_hbm, o_hbm)

    return combine_kernel(ywf, r1c, r2c)


def kernel(x, router_W, gate_W, up_W, down_W):
    Bx, Tx, Dx = x.shape
    N = Bx * Tx
    xf = x.reshape(N, Dx)

    logits = (xf @ router_W.T).astype(jnp.float32)
    idx1 = jnp.argmax(logits, axis=-1).astype(jnp.int32)
    m1 = jnp.max(logits, axis=-1)
    eids = jnp.arange(E, dtype=jnp.int32)
    masked = jnp.where(idx1[:, None] == eids[None, :], -jnp.inf, logits)
    idx2 = jnp.argmax(masked, axis=-1).astype(jnp.int32)
    m2 = jnp.max(masked, axis=-1)
    e2 = jnp.exp(m2 - m1)
    w1 = 1.0 / (1.0 + e2)
    w2 = e2 * w1

    pairs_e = jnp.concatenate([idx1, idx2])
    onehot = (pairs_e[:, None] == eids[None, :]).astype(jnp.int32)
    ranks = jnp.cumsum(onehot, axis=0) - onehot
    rank = jnp.sum(ranks * onehot, axis=1)
    counts = onehot.sum(axis=0)
    blocks = (counts + M - 1) // M
    cum_blocks = jnp.cumsum(blocks)
    padded_start = (cum_blocks - blocks) * M
    dest = padded_start[pairs_e] + rank

    pairs_t = jnp.tile(jnp.arange(N, dtype=jnp.int32), K)
    token_of_row = jnp.zeros((NRP,), jnp.int32).at[dest].set(pairs_t)
    block_expert = jnp.searchsorted(
        cum_blocks, jnp.arange(NB, dtype=jnp.int32), side='right')
    block_expert = jnp.minimum(block_expert, E - 1).astype(jnp.int32)

    nact = cum_blocks[-1:].astype(jnp.int32)
    w_col = jnp.zeros((NRP,), jnp.float32).at[dest].set(
        jnp.concatenate([w1, w2]).astype(jnp.float32)).reshape(NRP, 1)
    xg = xf.astype(jnp.bfloat16)[token_of_row]
    yw = _grouped_ffn(block_expert, nact, xg, gate_W, up_W, down_W, w_col)

    out = _sc_combine(yw, dest[:N], dest[N:], N)
    return out.reshape(Bx, Tx, Dx).astype(x.dtype)


# speedup vs baseline: 1.0002x; 1.0002x over previous
import functools

import jax
import jax.numpy as jnp
from jax.experimental import pallas as pl
from jax.experimental.pallas import tpu as pltpu
from jax.experimental.pallas import tpu_sc as plsc

D = 1024
E = 8
K = 2
H = 4096
M = 512
NB = 23
NRP = NB * M
HB = 512
NH = H // HB


def _ffn_kernel(be_ref, na_ref, xg_ref, gate_ref, up_ref, down_ref, w_ref,
                o_ref, acc_ref):
    b = pl.program_id(0)
    h = pl.program_id(1)

    @pl.when(b < na_ref[0])
    def _():
        xg = xg_ref[...]
        g = jax.lax.dot_general(
            xg, gate_ref[0].astype(jnp.bfloat16), (((1,), (1,)), ((), ())),
            preferred_element_type=jnp.float32)
        u = jax.lax.dot_general(
            xg, up_ref[0].astype(jnp.bfloat16), (((1,), (1,)), ((), ())),
            preferred_element_type=jnp.float32)
        hact = (g * jax.nn.sigmoid(g) * u).astype(jnp.bfloat16)
        yp = jax.lax.dot_general(
            hact, down_ref[0].astype(jnp.bfloat16), (((1,), (1,)), ((), ())),
            preferred_element_type=jnp.float32)

        @pl.when(h == 0)
        def _():
            acc_ref[...] = yp

        @pl.when(h > 0)
        def _():
            acc_ref[...] += yp

        @pl.when(h == NH - 1)
        def _():
            o_ref[...] = acc_ref[...] * w_ref[...]


def _grouped_ffn(block_expert, nact, xg, gate_W, up_W, down_W, w_col):
    def _row_ix(b, h, be, na):
        return (jnp.where(b < na[0], b, jnp.minimum(na[0], NB - 1)), 0)

    def _gu_ix(b, h, be, na):
        return (be[b], jnp.where(b < na[0], h, 0), 0)

    def _dn_ix(b, h, be, na):
        return (be[b], 0, jnp.where(b < na[0], h, 0))

    grid_spec = pltpu.PrefetchScalarGridSpec(
        num_scalar_prefetch=2,
        grid=(NB, NH),
        in_specs=[
            pl.BlockSpec((M, D), _row_ix),
            pl.BlockSpec((1, HB, D), _gu_ix),
            pl.BlockSpec((1, HB, D), _gu_ix),
            pl.BlockSpec((1, D, HB), _dn_ix),
            pl.BlockSpec((M, 1), _row_ix),
        ],
        out_specs=pl.BlockSpec((M, D), _row_ix),
        scratch_shapes=[pltpu.VMEM((M, D), jnp.float32)],
    )
    return pl.pallas_call(
        _ffn_kernel,
        grid_spec=grid_spec,
        out_shape=jax.ShapeDtypeStruct((NRP, D), jnp.float32),
    )(block_expert, nact, xg, gate_W, up_W, down_W, w_col)


GW = 128
DT = 128
NDT = D // DT


def _sc_combine(yw, r1, r2, n_tok):
    ywf = yw.reshape(NRP * NDT, DT)
    cid = jnp.arange(NDT, dtype=jnp.int32)[:, None]
    r1c = r1[None, :] * NDT + cid
    r2c = r2[None, :] * NDT + cid
    mesh = plsc.VectorSubcoreMesh(core_axis_name="c", subcore_axis_name="s")

    @functools.partial(
        pl.kernel,
        out_type=jax.ShapeDtypeStruct((n_tok, D), jnp.float32),
        mesh=mesh,
        scratch_types=[pltpu.VMEM((GW, DT), jnp.float32),
                       pltpu.VMEM((GW, DT), jnp.float32)])
    def combine_kernel(ywf_hbm, r1_hbm, r2_hbm, o_hbm, g1, g2):
        def body(r1_vmem, r2_vmem, o_vmem):
            pltpu.sync_copy(ywf_hbm.at[r1_vmem.at[0]], g1)
            pltpu.sync_copy(ywf_hbm.at[r2_vmem.at[0]], g2)

            @pl.loop(0, GW)
            def _(i):
                for j in range(0, DT, 16):
                    slc = (pl.ds(i, 1), pl.ds(j, 16))
                    o_vmem.at[slc][...] = g1.at[slc][...] + g2.at[slc][...]

        pltpu.emit_pipeline(
            body,
            grid=(n_tok // GW, NDT),
            in_specs=[pl.BlockSpec((1, GW), lambda i, c: (c, i)),
                      pl.BlockSpec((1, GW), lambda i, c: (c, i))],
            out_specs=[pl.BlockSpec((GW, DT), lambda i, c: (i, c))],
            core_axis_name=("c", "s"),
            dimension_semantics=(pltpu.PARALLEL, pltpu.PARALLEL),
        )(r1_hbm, r2_hbm, o_hbm)

    return combine_kernel(ywf, r1c, r2c)


def kernel(x, router_W, gate_W, up_W, down_W):
    Bx, Tx, Dx = x.shape
    N = Bx * Tx
    xf = x.reshape(N, Dx)

    logits = (xf @ router_W.T).astype(jnp.float32)
    idx1 = jnp.argmax(logits, axis=-1).astype(jnp.int32)
    m1 = jnp.max(logits, axis=-1)
    eids = jnp.arange(E, dtype=jnp.int32)
    masked = jnp.where(idx1[:, None] == eids[None, :], -jnp.inf, logits)
    idx2 = jnp.argmax(masked, axis=-1).astype(jnp.int32)
    m2 = jnp.max(masked, axis=-1)
    e2 = jnp.exp(m2 - m1)
    w1 = 1.0 / (1.0 + e2)
    w2 = e2 * w1

    pairs_e = jnp.concatenate([idx1, idx2])
    onehot = (pairs_e[:, None] == eids[None, :]).astype(jnp.int32)
    ranks = jnp.cumsum(onehot, axis=0) - onehot
    rank = jnp.sum(ranks * onehot, axis=1)
    counts = onehot.sum(axis=0)
    blocks = (counts + M - 1) // M
    cum_blocks = jnp.cumsum(blocks)
    padded_start = (cum_blocks - blocks) * M
    dest = padded_start[pairs_e] + rank

    pairs_t = jnp.tile(jnp.arange(N, dtype=jnp.int32), K)
    token_of_row = jnp.zeros((NRP,), jnp.int32).at[dest].set(pairs_t)
    block_expert = jnp.searchsorted(
        cum_blocks, jnp.arange(NB, dtype=jnp.int32), side='right')
    block_expert = jnp.minimum(block_expert, E - 1).astype(jnp.int32)

    nact = cum_blocks[-1:].astype(jnp.int32)
    w_col = jnp.zeros((NRP,), jnp.float32).at[dest].set(
        jnp.concatenate([w1, w2]).astype(jnp.float32)).reshape(NRP, 1)
    xg = xf.astype(jnp.bfloat16)[token_of_row]
    yw = _grouped_ffn(block_expert, nact, xg, gate_W, up_W, down_W, w_col)

    out = _sc_combine(yw, dest[:N], dest[N:], N)
    return out.reshape(Bx, Tx, Dx).astype(x.dtype)


# speedup vs baseline: 1.0782x; 1.0780x over previous
import functools

import jax
import jax.numpy as jnp
from jax.experimental import pallas as pl
from jax.experimental.pallas import tpu as pltpu
from jax.experimental.pallas import tpu_sc as plsc

D = 1024
E = 8
K = 2
H = 4096
M = 512
NB = 23
NRP = NB * M
HB = 512
NH = H // HB


def _ffn_kernel(be_ref, na_ref, xg_ref, gate_ref, up_ref, down_ref, w_ref,
                o_ref, acc_ref):
    b = pl.program_id(0)
    h = pl.program_id(1)

    @pl.when(b < na_ref[0])
    def _():
        xg = xg_ref[...]
        g = jax.lax.dot_general(
            xg, gate_ref[0].astype(jnp.bfloat16), (((1,), (1,)), ((), ())),
            preferred_element_type=jnp.float32)
        u = jax.lax.dot_general(
            xg, up_ref[0].astype(jnp.bfloat16), (((1,), (1,)), ((), ())),
            preferred_element_type=jnp.float32)
        hact = (g * jax.nn.sigmoid(g) * u).astype(jnp.bfloat16)
        yp = jax.lax.dot_general(
            hact, down_ref[0].astype(jnp.bfloat16), (((1,), (1,)), ((), ())),
            preferred_element_type=jnp.float32)

        @pl.when(h == 0)
        def _():
            acc_ref[...] = yp

        @pl.when(h > 0)
        def _():
            acc_ref[...] += yp

        @pl.when(h == NH - 1)
        def _():
            o_ref[...] = jnp.reshape(acc_ref[...] * w_ref[...],
                                     (M * NDT, DT))


def _grouped_ffn(block_expert, nact, xg, gate_W, up_W, down_W, w_col):
    def _row_ix(b, h, be, na):
        return (jnp.where(b < na[0], b, jnp.minimum(na[0], NB - 1)), 0)

    def _gu_ix(b, h, be, na):
        return (be[b], jnp.where(b < na[0], h, 0), 0)

    def _dn_ix(b, h, be, na):
        return (be[b], 0, jnp.where(b < na[0], h, 0))

    grid_spec = pltpu.PrefetchScalarGridSpec(
        num_scalar_prefetch=2,
        grid=(NB, NH),
        in_specs=[
            pl.BlockSpec((M, D), _row_ix),
            pl.BlockSpec((1, HB, D), _gu_ix),
            pl.BlockSpec((1, HB, D), _gu_ix),
            pl.BlockSpec((1, D, HB), _dn_ix),
            pl.BlockSpec((M, 1), _row_ix),
        ],
        out_specs=pl.BlockSpec((M * NDT, DT), _row_ix),
        scratch_shapes=[pltpu.VMEM((M, D), jnp.float32)],
    )
    return pl.pallas_call(
        _ffn_kernel,
        grid_spec=grid_spec,
        out_shape=jax.ShapeDtypeStruct((NRP * NDT, DT), jnp.float32),
    )(block_expert, nact, xg, gate_W, up_W, down_W, w_col)


GW = 128
DT = 128
NDT = D // DT


def _sc_combine(yw, r1, r2, n_tok):
    cid = jnp.arange(NDT, dtype=jnp.int32)[:, None]
    r1c = r1[None, :] * NDT + cid
    r2c = r2[None, :] * NDT + cid
    mesh = plsc.VectorSubcoreMesh(core_axis_name="c", subcore_axis_name="s")

    @functools.partial(
        pl.kernel,
        out_type=jax.ShapeDtypeStruct((n_tok, D), jnp.float32),
        mesh=mesh,
        scratch_types=[pltpu.VMEM((GW, DT), jnp.float32),
                       pltpu.VMEM((GW, DT), jnp.float32),
                       pltpu.SemaphoreType.DMA,
                       pltpu.SemaphoreType.DMA])
    def combine_kernel(ywf_hbm, r1_hbm, r2_hbm, o_hbm, g1, g2, sem1, sem2):
        def body(r1_vmem, r2_vmem, o_vmem):
            c1 = pltpu.async_copy(ywf_hbm.at[r1_vmem.at[0]], g1, sem1)
            c2 = pltpu.async_copy(ywf_hbm.at[r2_vmem.at[0]], g2, sem2)
            c1.wait()
            c2.wait()

            @pl.loop(0, GW)
            def _(i):
                for j in range(0, DT, 16):
                    slc = (pl.ds(i, 1), pl.ds(j, 16))
                    o_vmem.at[slc][...] = g1.at[slc][...] + g2.at[slc][...]

        pltpu.emit_pipeline(
            body,
            grid=(n_tok // GW, NDT),
            in_specs=[pl.BlockSpec((1, GW), lambda i, c: (c, i)),
                      pl.BlockSpec((1, GW), lambda i, c: (c, i))],
            out_specs=[pl.BlockSpec((GW, DT), lambda i, c: (i, c))],
            core_axis_name=("c", "s"),
            dimension_semantics=(pltpu.PARALLEL, pltpu.PARALLEL),
        )(r1_hbm, r2_hbm, o_hbm)

    return combine_kernel(yw, r1c, r2c)


def kernel(x, router_W, gate_W, up_W, down_W):
    Bx, Tx, Dx = x.shape
    N = Bx * Tx
    xf = x.reshape(N, Dx)

    logits = (xf @ router_W.T).astype(jnp.float32)
    idx1 = jnp.argmax(logits, axis=-1).astype(jnp.int32)
    m1 = jnp.max(logits, axis=-1)
    eids = jnp.arange(E, dtype=jnp.int32)
    masked = jnp.where(idx1[:, None] == eids[None, :], -jnp.inf, logits)
    idx2 = jnp.argmax(masked, axis=-1).astype(jnp.int32)
    m2 = jnp.max(masked, axis=-1)
    e2 = jnp.exp(m2 - m1)
    w1 = 1.0 / (1.0 + e2)
    w2 = e2 * w1

    pairs_e = jnp.concatenate([idx1, idx2])
    onehot = (pairs_e[:, None] == eids[None, :]).astype(jnp.int32)
    ranks = jnp.cumsum(onehot, axis=0) - onehot
    rank = jnp.sum(ranks * onehot, axis=1)
    counts = onehot.sum(axis=0)
    blocks = (counts + M - 1) // M
    cum_blocks = jnp.cumsum(blocks)
    padded_start = (cum_blocks - blocks) * M
    dest = padded_start[pairs_e] + rank

    pairs_t = jnp.tile(jnp.arange(N, dtype=jnp.int32), K)
    token_of_row = jnp.zeros((NRP,), jnp.int32).at[dest].set(pairs_t)
    block_expert = jnp.searchsorted(
        cum_blocks, jnp.arange(NB, dtype=jnp.int32), side='right')
    block_expert = jnp.minimum(block_expert, E - 1).astype(jnp.int32)

    nact = cum_blocks[-1:].astype(jnp.int32)
    w_col = jnp.zeros((NRP,), jnp.float32).at[dest].set(
        jnp.concatenate([w1, w2]).astype(jnp.float32)).reshape(NRP, 1)
    xg = xf.astype(jnp.bfloat16)[token_of_row]
    yw = _grouped_ffn(block_expert, nact, xg, gate_W, up_W, down_W, w_col)

    out = _sc_combine(yw, dest[:N], dest[N:], N)
    return out.reshape(Bx, Tx, Dx).astype(x.dtype)
